# baseline (speedup 1.0000x reference)
import jax
import jax.numpy as jnp
from jax.experimental import pallas as pl
from jax.experimental.pallas import tpu as pltpu

_BM = 2000


def _chain_kernel(x_ref, halo_ref, wcat_ref, w1_ref, w2_ref, wfc_ref,
                  bt_ref, b1_ref, bf_ref, out_ref):
    cur = x_ref[...]
    bm = cur.shape[0]
    rowid = jax.lax.broadcasted_iota(jnp.int32, cur.shape, 0)
    xm1 = pltpu.roll(cur, shift=1, axis=0)
    xm1 = jnp.where(rowid == 0, halo_ref[0, 0:1, :], xm1)
    xp1 = pltpu.roll(cur, shift=bm - 1, axis=0)
    xp1 = jnp.where(rowid == bm - 1, halo_ref[0, 1:2, :], xp1)
    xin = jnp.concatenate([xm1, cur, xp1], axis=1)
    y = jnp.dot(xin, wcat_ref[...], preferred_element_type=jnp.float32)
    y = jax.nn.relu(y + bt_ref[...])
    z = jnp.dot(y, w1_ref[...], preferred_element_type=jnp.float32)
    z = jax.nn.relu(z + b1_ref[...])
    wf = jnp.dot(w2_ref[...], wfc_ref[...], preferred_element_type=jnp.float32)
    z = jnp.dot(z, wf, preferred_element_type=jnp.float32) + bf_ref[...]
    out_ref[...] = z


def kernel(x, edge_index, edge_weights, W_t, b_t, W1, b1, W2, b2, W_fc, b_fc):
    B, T, N, C = x.shape
    H = W1.shape[0]
    C_OUT = W_fc.shape[1]
    J = N // _BM
    rows = B * N

    Wcat = jnp.concatenate(
        [W_t[:, :, 0, 0].T, W_t[:, :, 1, 0].T, W_t[:, :, 2, 0].T], axis=0)
    bf = (b2 @ W_fc + b_fc).reshape(1, C_OUT)

    xl = x[:, T - 1]

    zrow = jnp.zeros((B, 1, C), dtype=x.dtype)
    prev_rows = jnp.concatenate([zrow, xl[:, _BM - 1::_BM][:, :-1]], axis=1)
    next_rows = jnp.concatenate([xl[:, _BM::_BM], zrow], axis=1)
    halo = jnp.stack([prev_rows, next_rows], axis=2).reshape(B * J, 2, C)

    out = pl.pallas_call(
        _chain_kernel,
        grid=(rows // _BM,),
        in_specs=[
            pl.BlockSpec((_BM, C), lambda k: ((k // J * T + T - 1) * J + k % J, 0)),
            pl.BlockSpec((1, 2, C), lambda k: (k, 0, 0)),
            pl.BlockSpec((3 * C, H), lambda k: (0, 0)),
            pl.BlockSpec((H, H), lambda k: (0, 0)),
            pl.BlockSpec((H, H), lambda k: (0, 0)),
            pl.BlockSpec((H, C_OUT), lambda k: (0, 0)),
            pl.BlockSpec((1, H), lambda k: (0, 0)),
            pl.BlockSpec((1, H), lambda k: (0, 0)),
            pl.BlockSpec((1, C_OUT), lambda k: (0, 0)),
        ],
        out_specs=pl.BlockSpec((_BM, C_OUT), lambda k: (k, 0)),
        out_shape=jax.ShapeDtypeStruct((rows, C_OUT), jnp.float32),
    )(x.reshape(B * T * N, C), halo, Wcat, W1, W2, W_fc,
      b_t.reshape(1, H), b1.reshape(1, H), bf)
    return out.reshape(B, N, C_OUT)


# speedup vs baseline: 1.4315x; 1.4315x over previous
import jax
import jax.numpy as jnp
from jax.experimental import pallas as pl
from jax.experimental.pallas import tpu as pltpu

_BM = 1000


def _chain_kernel(x_ref, halo_ref, wcat_ref, w1_ref, w2_ref, wfc_ref,
                  bt_ref, b1_ref, bf_ref, out_ref):
    cur = x_ref[...]
    bm = cur.shape[0]
    rowid = jax.lax.broadcasted_iota(jnp.int32, cur.shape, 0)
    xm1 = pltpu.roll(cur, shift=1, axis=0)
    xm1 = jnp.where(rowid == 0, halo_ref[0, 0:1, :], xm1)
    xp1 = pltpu.roll(cur, shift=bm - 1, axis=0)
    xp1 = jnp.where(rowid == bm - 1, halo_ref[0, 1:2, :], xp1)
    xin = jnp.concatenate([xm1, cur, xp1], axis=1)
    y = jnp.dot(xin, wcat_ref[...], preferred_element_type=jnp.float32)
    y = jax.nn.relu(y + bt_ref[...])
    z = jnp.dot(y, w1_ref[...], preferred_element_type=jnp.float32)
    z = jax.nn.relu(z + b1_ref[...])
    wf = jnp.dot(w2_ref[...], wfc_ref[...], preferred_element_type=jnp.float32)
    z = jnp.dot(z, wf, preferred_element_type=jnp.float32) + bf_ref[...]
    out_ref[...] = z


def kernel(x, edge_index, edge_weights, W_t, b_t, W1, b1, W2, b2, W_fc, b_fc):
    B, T, N, C = x.shape
    H = W1.shape[0]
    C_OUT = W_fc.shape[1]
    J = N // _BM
    rows = B * N

    Wcat = jnp.concatenate(
        [W_t[:, :, 0, 0].T, W_t[:, :, 1, 0].T, W_t[:, :, 2, 0].T], axis=0)
    bf = (b2 @ W_fc + b_fc).reshape(1, C_OUT)

    xl = x[:, T - 1]

    zrow = jnp.zeros((B, 1, C), dtype=x.dtype)
    prev_rows = jnp.concatenate([zrow, xl[:, _BM - 1::_BM][:, :-1]], axis=1)
    next_rows = jnp.concatenate([xl[:, _BM::_BM], zrow], axis=1)
    halo = jnp.stack([prev_rows, next_rows], axis=2).reshape(B * J, 2, C)

    out = pl.pallas_call(
        _chain_kernel,
        grid=(rows // _BM,),
        in_specs=[
            pl.BlockSpec((_BM, C), lambda k: (k, 0)),
            pl.BlockSpec((1, 2, C), lambda k: (k, 0, 0)),
            pl.BlockSpec((3 * C, H), lambda k: (0, 0)),
            pl.BlockSpec((H, H), lambda k: (0, 0)),
            pl.BlockSpec((H, H), lambda k: (0, 0)),
            pl.BlockSpec((H, C_OUT), lambda k: (0, 0)),
            pl.BlockSpec((1, H), lambda k: (0, 0)),
            pl.BlockSpec((1, H), lambda k: (0, 0)),
            pl.BlockSpec((1, C_OUT), lambda k: (0, 0)),
        ],
        out_specs=pl.BlockSpec((_BM, C_OUT), lambda k: (k, 0)),
        out_shape=jax.ShapeDtypeStruct((rows, C_OUT), jnp.float32),
    )(xl.reshape(rows, C), halo, Wcat, W1, W2, W_fc,
      b_t.reshape(1, H), b1.reshape(1, H), bf)
    return out.reshape(B, N, C_OUT)


# speedup vs baseline: 1.9328x; 1.3501x over previous
import jax
import jax.numpy as jnp
from jax.experimental import pallas as pl
from jax.experimental.pallas import tpu as pltpu

_BM = 5000


def _chain_kernel(x_ref, halo_ref, wcat_ref, w1_ref, w2_ref, wfc_ref,
                  bt_ref, b1_ref, bf_ref, out_ref):
    cur = x_ref[...]
    bm = cur.shape[0]
    rowid = jax.lax.broadcasted_iota(jnp.int32, cur.shape, 0)
    xm1 = pltpu.roll(cur, shift=1, axis=0)
    xm1 = jnp.where(rowid == 0, halo_ref[0, 0:1, :], xm1)
    xp1 = pltpu.roll(cur, shift=bm - 1, axis=0)
    xp1 = jnp.where(rowid == bm - 1, halo_ref[0, 1:2, :], xp1)
    xin = jnp.concatenate([xm1, cur, xp1], axis=1)
    y = jnp.dot(xin, wcat_ref[...], preferred_element_type=jnp.float32)
    y = jax.nn.relu(y + bt_ref[...])
    z = jnp.dot(y, w1_ref[...], preferred_element_type=jnp.float32)
    z = jax.nn.relu(z + b1_ref[...])
    wf = jnp.dot(w2_ref[...], wfc_ref[...], preferred_element_type=jnp.float32)
    z = jnp.dot(z, wf, preferred_element_type=jnp.float32) + bf_ref[...]
    out_ref[...] = z


def kernel(x, edge_index, edge_weights, W_t, b_t, W1, b1, W2, b2, W_fc, b_fc):
    B, T, N, C = x.shape
    H = W1.shape[0]
    C_OUT = W_fc.shape[1]
    J = N // _BM
    rows = B * N

    Wcat = jnp.concatenate(
        [W_t[:, :, 0, 0].T, W_t[:, :, 1, 0].T, W_t[:, :, 2, 0].T], axis=0)
    bf = (b2 @ W_fc + b_fc).reshape(1, C_OUT)

    xl = x[:, T - 1]

    zrow = jnp.zeros((B, 1, C), dtype=x.dtype)
    prev_rows = jnp.concatenate([zrow, xl[:, _BM - 1::_BM][:, :-1]], axis=1)
    next_rows = jnp.concatenate([xl[:, _BM::_BM], zrow], axis=1)
    halo = jnp.stack([prev_rows, next_rows], axis=2).reshape(B * J, 2, C)

    out = pl.pallas_call(
        _chain_kernel,
        grid=(rows // _BM,),
        in_specs=[
            pl.BlockSpec((_BM, C), lambda k: (k, 0)),
            pl.BlockSpec((1, 2, C), lambda k: (k, 0, 0)),
            pl.BlockSpec((3 * C, H), lambda k: (0, 0)),
            pl.BlockSpec((H, H), lambda k: (0, 0)),
            pl.BlockSpec((H, H), lambda k: (0, 0)),
            pl.BlockSpec((H, C_OUT), lambda k: (0, 0)),
            pl.BlockSpec((1, H), lambda k: (0, 0)),
            pl.BlockSpec((1, H), lambda k: (0, 0)),
            pl.BlockSpec((1, C_OUT), lambda k: (0, 0)),
        ],
        out_specs=pl.BlockSpec((_BM, C_OUT), lambda k: (k, 0)),
        out_shape=jax.ShapeDtypeStruct((rows, C_OUT), jnp.float32),
    )(xl.reshape(rows, C), halo, Wcat, W1, W2, W_fc,
      b_t.reshape(1, H), b1.reshape(1, H), bf)
    return out.reshape(B, N, C_OUT)


# speedup vs baseline: 2.1715x; 1.1235x over previous
import jax
import jax.numpy as jnp
from jax.experimental import pallas as pl
from jax.experimental.pallas import tpu as pltpu

_BM = 10000


def _chain_kernel(x_ref, halo_ref, wcat_ref, w1_ref, w2_ref, wfc_ref,
                  bt_ref, b1_ref, bf_ref, out_ref):
    cur = x_ref[...]
    bm = cur.shape[0]
    rowid = jax.lax.broadcasted_iota(jnp.int32, cur.shape, 0)
    xm1 = pltpu.roll(cur, shift=1, axis=0)
    xm1 = jnp.where(rowid == 0, halo_ref[0, 0:1, :], xm1)
    xp1 = pltpu.roll(cur, shift=bm - 1, axis=0)
    xp1 = jnp.where(rowid == bm - 1, halo_ref[0, 1:2, :], xp1)
    xin = jnp.concatenate([xm1, cur, xp1], axis=1)
    y = jnp.dot(xin, wcat_ref[...], preferred_element_type=jnp.float32)
    y = jax.nn.relu(y + bt_ref[...])
    z = jnp.dot(y, w1_ref[...], preferred_element_type=jnp.float32)
    z = jax.nn.relu(z + b1_ref[...])
    wf = jnp.dot(w2_ref[...], wfc_ref[...], preferred_element_type=jnp.float32)
    z = jnp.dot(z, wf, preferred_element_type=jnp.float32) + bf_ref[...]
    out_ref[...] = z


def kernel(x, edge_index, edge_weights, W_t, b_t, W1, b1, W2, b2, W_fc, b_fc):
    B, T, N, C = x.shape
    H = W1.shape[0]
    C_OUT = W_fc.shape[1]
    J = N // _BM
    rows = B * N

    Wcat = jnp.concatenate(
        [W_t[:, :, 0, 0].T, W_t[:, :, 1, 0].T, W_t[:, :, 2, 0].T], axis=0)
    bf = (b2 @ W_fc + b_fc).reshape(1, C_OUT)

    xl = x[:, T - 1]

    zrow = jnp.zeros((B, 1, C), dtype=x.dtype)
    prev_rows = jnp.concatenate([zrow, xl[:, _BM - 1::_BM][:, :-1]], axis=1)
    next_rows = jnp.concatenate([xl[:, _BM::_BM], zrow], axis=1)
    halo = jnp.stack([prev_rows, next_rows], axis=2).reshape(B * J, 2, C)

    out = pl.pallas_call(
        _chain_kernel,
        grid=(rows // _BM,),
        in_specs=[
            pl.BlockSpec((_BM, C), lambda k: (k, 0)),
            pl.BlockSpec((1, 2, C), lambda k: (k, 0, 0)),
            pl.BlockSpec((3 * C, H), lambda k: (0, 0)),
            pl.BlockSpec((H, H), lambda k: (0, 0)),
            pl.BlockSpec((H, H), lambda k: (0, 0)),
            pl.BlockSpec((H, C_OUT), lambda k: (0, 0)),
            pl.BlockSpec((1, H), lambda k: (0, 0)),
            pl.BlockSpec((1, H), lambda k: (0, 0)),
            pl.BlockSpec((1, C_OUT), lambda k: (0, 0)),
        ],
        out_specs=pl.BlockSpec((_BM, C_OUT), lambda k: (k, 0)),
        out_shape=jax.ShapeDtypeStruct((rows, C_OUT), jnp.float32),
    )(xl.reshape(rows, C), halo, Wcat, W1, W2, W_fc,
      b_t.reshape(1, H), b1.reshape(1, H), bf)
    return out.reshape(B, N, C_OUT)
